# baseline (speedup 1.0000x reference)
import functools

import jax
import jax.numpy as jnp
from jax import lax
from jax.experimental import pallas as pl
from jax.experimental.pallas import tpu as pltpu
from jax.experimental.pallas import tpu_sc as plsc

N_MAT = 10000
N_OP = 10000
E = 320000
EMB = 128
EA = 16

NC = 2
NS = 16
NW = NC * NS

BLK_E = 6400
DPW = EMB // NW
EB = 8000
NB = E // EB
CH = E // NW
LANES = 16

_SC_MESH = dict(core_axis_name="c", subcore_axis_name="s",
                num_cores=NC, num_subcores=NS)
_SC_PARAMS = pltpu.CompilerParams(needs_layout_passes=False)

_HI = lax.Precision.HIGHEST


def _eye(n):
    r = lax.broadcasted_iota(jnp.int32, (n, n), 0)
    c = lax.broadcasted_iota(jnp.int32, (n, n), 1)
    return jnp.where(r == c, 1.0, 0.0).astype(jnp.float32)


def _node_body(mat_ref, ops_ref, wmat_ref, wop_ref, asc_ref, ac_ref,
               up_ref, opst_ref, sself_ref, smat_ref, sop_ref):
    dn = (((1,), (1,)), ((), ()))
    up = lax.dot_general(mat_ref[...], wmat_ref[...], dn,
                         preferred_element_type=jnp.float32)
    opsup = lax.dot_general(ops_ref[...], wop_ref[...], dn,
                            preferred_element_type=jnp.float32)
    up_ref[...] = up
    opst_ref[...] = lax.dot_general(_eye(EMB), opsup, (((1,), (1,)), ((), ())),
                                    precision=_HI,
                                    preferred_element_type=jnp.float32)
    bf = lambda x: x.astype(jnp.bfloat16).astype(jnp.float32)
    upb = bf(up)
    opsupb = bf(opsup)
    c_sum = bf(asc_ref[0:128, 0]) + bf(asc_ref[128:256, 0])
    b1 = bf(ac_ref[0:128, 0])
    b2 = bf(ac_ref[128:256, 0])
    s = jnp.sum(upb * c_sum[None, :], axis=1, keepdims=True)
    sself_ref[...] = jnp.maximum(s, 0.2 * s)
    smat_ref[...] = jnp.sum(upb * b1[None, :], axis=1, keepdims=True)
    sop_ref[...] = jnp.sum(opsupb * b2[None, :], axis=1, keepdims=True)


def _node_stage(materials, ops_pad, W_mat, W_op, asc, ac):
    f32 = jnp.float32
    return pl.pallas_call(
        _node_body,
        out_shape=[
            jax.ShapeDtypeStruct((N_MAT, EMB), f32),
            jax.ShapeDtypeStruct((EMB, N_OP), f32),
            jax.ShapeDtypeStruct((N_MAT, 1), f32),
            jax.ShapeDtypeStruct((N_MAT, 1), f32),
            jax.ShapeDtypeStruct((N_OP, 1), f32),
        ],
    )(materials, ops_pad, W_mat, W_op, asc, ac)


PKR = E * EA // 128
BLK_P = 1600


def _sea_body(pk_ref, w2_ref, ac_ref, sea_ref):
    bf = lambda x: x.astype(jnp.bfloat16).astype(jnp.float32)
    b2 = bf(ac_ref[128:256, 0])
    wv = jnp.sum(b2[:, None] * bf(w2_ref[...]), axis=0)
    wvfull = jnp.concatenate([wv] * 8)
    jj = lax.broadcasted_iota(jnp.int32, (128, 8), 0)
    gg = lax.broadcasted_iota(jnp.int32, (128, 8), 1)
    wv_big = jnp.where(jj // 16 == gg, wvfull[:, None], 0.0)
    sea_ref[...] = lax.dot_general(bf(pk_ref[...]), wv_big,
                                   (((1,), (0,)), ((), ())),
                                   precision=_HI,
                                   preferred_element_type=jnp.float32)


def _sea_stage(ea_packed, W2, ac):
    grid = (PKR // BLK_P,)
    return pl.pallas_call(
        _sea_body,
        grid=grid,
        in_specs=[pl.BlockSpec((BLK_P, 128), lambda i: (i, 0)),
                  pl.BlockSpec((128, EA), lambda i: (0, 0)),
                  pl.BlockSpec((256, 1), lambda i: (0, 0))],
        out_specs=pl.BlockSpec((BLK_P, 8), lambda i: (i, 0)),
        out_shape=jax.ShapeDtypeStruct((PKR, 8), jnp.float32),
    )(ea_packed, W2, ac)


SB = 2000
NSB = CH // SB


def _cross_logit_body(smat_hbm, sop_hbm, src_hbm, dst_hbm, sea_hbm, eabp_hbm,
                      out_hbm, eat_hbm,
                      smat_v, sop_v, src_v, dst_v, sea_v, out_v,
                      pk_v, eb0, eb1, sem0, sem1):
    wid = lax.axis_index("s") * NC + lax.axis_index("c")
    base = wid * CH
    pltpu.sync_copy(smat_hbm, smat_v)
    pltpu.sync_copy(sop_hbm, sop_v)
    pltpu.sync_copy(src_hbm.at[pl.ds(base, CH)], src_v)
    pltpu.sync_copy(dst_hbm.at[pl.ds(base, CH)], dst_v)
    pltpu.sync_copy(sea_hbm.at[pl.ds(base, CH)], sea_v)

    @plsc.parallel_loop(0, CH // LANES, unroll=8)
    def body(i):
        sl = pl.ds(i * LANES, LANES)
        a = plsc.load_gather(sop_v, [src_v[sl]])
        b = plsc.load_gather(smat_v, [dst_v[sl]])
        x = a + b + sea_v[sl]
        out_v[sl] = jnp.maximum(x, 0.2 * x)

    pltpu.sync_copy(out_v, out_hbm.at[pl.ds(base, CH)])

    i16 = lax.iota(jnp.int32, LANES) * EA
    bufs = (eb0, eb1)
    sems = (sem0, sem1)
    pending = [None, None]
    w0 = wid * (CH * EA)
    for sb in range(NSB):
        pltpu.sync_copy(eabp_hbm.at[pl.ds(w0 + sb * (SB * EA), SB * EA)],
                        pk_v)
        for k in range(EA):
            slot = (sb * EA + k) % 2
            buf = bufs[slot]
            if pending[slot] is not None:
                pending[slot].wait()

            @plsc.parallel_loop(0, SB // LANES, unroll=8)
            def fill(i):
                flat = i16 + (i * (LANES * EA) + k)
                vals = plsc.load_gather(pk_v, [flat])
                buf[pl.ds(i * LANES, LANES)] = vals

            cp = pltpu.make_async_copy(
                buf, eat_hbm.at[pl.ds(k * E + base + sb * SB, SB)], sems[slot])
            cp.start()
            pending[slot] = cp
    for slot in range(2):
        if pending[slot] is not None:
            pending[slot].wait()


def _cross_logit_stage(s_mat, s_op, src, dst, s_ea, eabp):
    f32, i32 = jnp.float32, jnp.int32
    k = pl.kernel(
        _cross_logit_body,
        out_type=[jax.ShapeDtypeStruct((E,), f32),
                  jax.ShapeDtypeStruct((EA * E,), f32)],
        mesh=plsc.VectorSubcoreMesh(**_SC_MESH),
        compiler_params=_SC_PARAMS,
        scratch_types=[
            pltpu.VMEM((N_MAT,), f32),
            pltpu.VMEM((N_OP,), f32),
            pltpu.VMEM((CH,), i32),
            pltpu.VMEM((CH,), i32),
            pltpu.VMEM((CH,), f32),
            pltpu.VMEM((CH,), f32),
            pltpu.VMEM((SB * EA,), f32),
            pltpu.VMEM((SB,), f32),
            pltpu.VMEM((SB,), f32),
            pltpu.SemaphoreType.DMA,
            pltpu.SemaphoreType.DMA,
        ],
    )
    return k(s_mat, s_op, src, dst, s_ea, eabp)


def _softmax_body(a_ref, b_ref, wa_ref, wb_ref):
    a = a_ref[...]
    b = b_ref[...]
    m = jnp.maximum(jnp.max(a), jnp.max(b))
    ea_ = jnp.exp(a - m)
    eb_ = jnp.exp(b - m)
    inv = 1.0 / (jnp.sum(ea_) + jnp.sum(eb_))
    wa_ref[...] = ea_ * inv
    wb_ref[...] = eb_ * inv


def _softmax_stage(sself, clog2):
    return pl.pallas_call(
        _softmax_body,
        out_shape=[jax.ShapeDtypeStruct(sself.shape, jnp.float32),
                   jax.ShapeDtypeStruct(clog2.shape, jnp.float32)],
    )(sself, clog2)


def _scatter_body(src_hbm, dst_hbm, wc_hbm, opst_hbm, eat_hbm,
                  acct_hbm, seg16_hbm,
                  t0, t1, t2, t3, a0, a1, a2, a3, acc16,
                  src_v, dst_v, w_v, ea_v):
    wid = lax.axis_index("s") * NC + lax.axis_index("c")
    k_ea = wid % EA
    half = wid // EA
    tbls = (t0, t1, t2, t3)
    accs = (a0, a1, a2, a3)
    for j in range(DPW):
        pltpu.sync_copy(opst_hbm.at[pl.ds((wid * DPW + j) * N_OP, N_OP)],
                        tbls[j])

    zeros = jnp.zeros((LANES,), jnp.float32)
    for j in range(DPW):
        acc_j = accs[j]

        @plsc.parallel_loop(0, N_MAT // LANES, unroll=8)
        def z_body(i):
            acc_j[pl.ds(i * LANES, LANES)] = zeros

    @plsc.parallel_loop(0, N_MAT // LANES, unroll=8)
    def z16_body(i):
        acc16[pl.ds(i * LANES, LANES)] = zeros

    def outer(b, _):
        base = b * EB
        pltpu.sync_copy(src_hbm.at[pl.ds(base, EB)], src_v)
        pltpu.sync_copy(dst_hbm.at[pl.ds(base, EB)], dst_v)
        pltpu.sync_copy(wc_hbm.at[pl.ds(base, EB)], w_v)

        @plsc.parallel_loop(0, EB // LANES, unroll=8)
        def inner(i):
            sl = pl.ds(i * LANES, LANES)
            s16 = src_v[sl]
            d16 = dst_v[sl]
            wv = w_v[sl]
            for j in range(DPW):
                vals = plsc.load_gather(tbls[j], [s16])
                plsc.addupdate_scatter(accs[j], [d16], vals * wv)

        @pl.when((b % 2) == half)
        def _():
            pltpu.sync_copy(eat_hbm.at[pl.ds(k_ea * E + base, EB)], ea_v)

            @plsc.parallel_loop(0, EB // LANES, unroll=8)
            def inner_ea(i):
                sl = pl.ds(i * LANES, LANES)
                d16 = dst_v[sl]
                wv = w_v[sl]
                ev = ea_v[sl]
                plsc.addupdate_scatter(acc16, [d16], ev * wv)

        return 0

    lax.fori_loop(0, NB, outer, 0)
    for j in range(DPW):
        pltpu.sync_copy(accs[j],
                        acct_hbm.at[pl.ds((wid * DPW + j) * N_MAT, N_MAT)])
    pltpu.sync_copy(acc16, seg16_hbm.at[pl.ds(wid * N_MAT, N_MAT)])


def _scatter_stage(src, dst, wc, opst_flat, eat_flat):
    f32, i32 = jnp.float32, jnp.int32
    k = pl.kernel(
        _scatter_body,
        out_type=[jax.ShapeDtypeStruct((EMB * N_MAT,), f32),
                  jax.ShapeDtypeStruct((NW * N_MAT,), f32)],
        mesh=plsc.VectorSubcoreMesh(**_SC_MESH),
        compiler_params=_SC_PARAMS,
        scratch_types=(
            [pltpu.VMEM((N_OP,), f32)] * DPW
            + [pltpu.VMEM((N_MAT,), f32)] * DPW
            + [pltpu.VMEM((N_MAT,), f32),
               pltpu.VMEM((EB,), i32),
               pltpu.VMEM((EB,), i32),
               pltpu.VMEM((EB,), f32),
               pltpu.VMEM((EB,), f32)]
        ),
    )
    return k(src, dst, wc, opst_flat, eat_flat)


def _combine_body(up_ref, wself_ref, acct_ref, seg_ref, w2_ref, out_ref):
    acc = lax.dot_general(acct_ref[...], _eye(EMB), (((0,), (0,)), ((), ())),
                          precision=_HI,
                          preferred_element_type=jnp.float32)
    segt = lax.dot_general(seg_ref[...], _eye(NW), (((0,), (0,)), ((), ())),
                           precision=_HI,
                           preferred_element_type=jnp.float32)
    seg = segt[:, 0:EA] + segt[:, EA:2 * EA]
    term2 = lax.dot_general(seg, w2_ref[...], (((1,), (1,)), ((), ())),
                            precision=_HI,
                            preferred_element_type=jnp.float32)
    x = wself_ref[...] * up_ref[...] + acc + term2
    out_ref[...] = jnp.where(x > 0, x, jnp.exp(jnp.minimum(x, 0.0)) - 1.0)


def _combine_stage(up, wself, acct, seg16p, W2b):
    return pl.pallas_call(
        _combine_body,
        out_shape=jax.ShapeDtypeStruct((N_MAT, EMB), jnp.float32),
    )(up, wself, acct, seg16p, W2b)


def kernel(materials, operations, edge_index, edge_attr,
           W_mat, W_op, att_self_coef, att_coef):
    f32 = jnp.float32
    src = edge_index[0]
    dst = edge_index[1]
    W2 = W_op[:, 112:]
    W2b = W2.astype(jnp.bfloat16).astype(f32)
    eabp = edge_attr.astype(jnp.bfloat16).astype(f32).reshape(-1)
    ops_pad = jnp.pad(operations, ((0, 0), (0, EMB - 112)))

    up, opst, s_self, s_mat, s_op = _node_stage(
        materials, ops_pad, W_mat, W_op, att_self_coef, att_coef)
    s_ea = _sea_stage(edge_attr.reshape(PKR, 128), W2, att_coef)

    clog, eat_flat = _cross_logit_stage(s_mat[:, 0], s_op[:, 0], src, dst,
                                        s_ea.reshape(-1), eabp)

    wself, wc2 = _softmax_stage(s_self, clog.reshape(E // 128, 128))
    wc = wc2.reshape(-1)

    acct_flat, seg16p = _scatter_stage(src, dst, wc,
                                       opst.reshape(-1), eat_flat)
    acct = acct_flat.reshape(EMB, N_MAT)
    seg16p = seg16p.reshape(NW, N_MAT)

    return _combine_stage(up, wself, acct, seg16p, W2b)


# speedup vs baseline: 1.0502x; 1.0502x over previous
import functools

import jax
import jax.numpy as jnp
from jax import lax
from jax.experimental import pallas as pl
from jax.experimental.pallas import tpu as pltpu
from jax.experimental.pallas import tpu_sc as plsc

N_MAT = 10000
N_OP = 10000
E = 320000
EMB = 128
EA = 16

NC = 2
NS = 16
NW = NC * NS

BLK_E = 6400
DPW = EMB // NW
EB = 8000
NB = E // EB
CH = E // NW
LANES = 16

_SC_MESH = dict(core_axis_name="c", subcore_axis_name="s",
                num_cores=NC, num_subcores=NS)
_SC_PARAMS = pltpu.CompilerParams(needs_layout_passes=False)

_HI = lax.Precision.HIGHEST


def _eye(n):
    r = lax.broadcasted_iota(jnp.int32, (n, n), 0)
    c = lax.broadcasted_iota(jnp.int32, (n, n), 1)
    return jnp.where(r == c, 1.0, 0.0).astype(jnp.float32)


def _node_body(mat_ref, ops_ref, wmat_ref, wop_ref, asc_ref, ac_ref,
               up_ref, opst_ref, sself_ref, smat_ref, sop_ref):
    dn = (((1,), (1,)), ((), ()))
    up = lax.dot_general(mat_ref[...], wmat_ref[...], dn,
                         preferred_element_type=jnp.float32)
    opsup = lax.dot_general(ops_ref[...], wop_ref[...], dn,
                            preferred_element_type=jnp.float32)
    up_ref[...] = up
    opst_ref[...] = lax.dot_general(_eye(EMB), opsup, (((1,), (1,)), ((), ())),
                                    precision=_HI,
                                    preferred_element_type=jnp.float32)
    bf = lambda x: x.astype(jnp.bfloat16).astype(jnp.float32)
    upb = bf(up)
    opsupb = bf(opsup)
    c_sum = bf(asc_ref[0:128, 0]) + bf(asc_ref[128:256, 0])
    b1 = bf(ac_ref[0:128, 0])
    b2 = bf(ac_ref[128:256, 0])
    s = jnp.sum(upb * c_sum[None, :], axis=1, keepdims=True)
    sself_ref[...] = jnp.maximum(s, 0.2 * s)
    smat_ref[...] = jnp.sum(upb * b1[None, :], axis=1, keepdims=True)
    sop_ref[...] = jnp.sum(opsupb * b2[None, :], axis=1, keepdims=True)


def _node_stage(materials, ops_pad, W_mat, W_op, asc, ac):
    f32 = jnp.float32
    return pl.pallas_call(
        _node_body,
        out_shape=[
            jax.ShapeDtypeStruct((N_MAT, EMB), f32),
            jax.ShapeDtypeStruct((EMB, N_OP), f32),
            jax.ShapeDtypeStruct((N_MAT, 1), f32),
            jax.ShapeDtypeStruct((N_MAT, 1), f32),
            jax.ShapeDtypeStruct((N_OP, 1), f32),
        ],
    )(materials, ops_pad, W_mat, W_op, asc, ac)


PKR = E * EA // 128
BLK_P = 1600


def _sea_body(pk_ref, w2_ref, ac_ref, sea_ref):
    bf = lambda x: x.astype(jnp.bfloat16).astype(jnp.float32)
    b2 = bf(ac_ref[128:256, 0])
    wv = jnp.sum(b2[:, None] * bf(w2_ref[...]), axis=0)
    wvfull = jnp.concatenate([wv] * 8)
    jj = lax.broadcasted_iota(jnp.int32, (128, 8), 0)
    gg = lax.broadcasted_iota(jnp.int32, (128, 8), 1)
    wv_big = jnp.where(jj // 16 == gg, wvfull[:, None], 0.0)
    sea_ref[...] = lax.dot_general(bf(pk_ref[...]), wv_big,
                                   (((1,), (0,)), ((), ())),
                                   precision=_HI,
                                   preferred_element_type=jnp.float32)


def _sea_stage(ea_packed, W2, ac):
    grid = (PKR // BLK_P,)
    return pl.pallas_call(
        _sea_body,
        grid=grid,
        in_specs=[pl.BlockSpec((BLK_P, 128), lambda i: (i, 0)),
                  pl.BlockSpec((128, EA), lambda i: (0, 0)),
                  pl.BlockSpec((256, 1), lambda i: (0, 0))],
        out_specs=pl.BlockSpec((BLK_P, 8), lambda i: (i, 0)),
        out_shape=jax.ShapeDtypeStruct((PKR, 8), jnp.float32),
    )(ea_packed, W2, ac)


SB = 2000
NSB = CH // SB


def _cross_logit_body(smat_hbm, sop_hbm, src_hbm, dst_hbm, sea_hbm, eabp_hbm,
                      out_hbm, eat_hbm,
                      smat_v, sop_v, src_v, dst_v, sea_v, out_v,
                      pk_v, eb0, sem0):
    wid = lax.axis_index("s") * NC + lax.axis_index("c")
    base = wid * CH
    pltpu.sync_copy(smat_hbm, smat_v)
    pltpu.sync_copy(sop_hbm, sop_v)
    pltpu.sync_copy(src_hbm.at[pl.ds(base, CH)], src_v)
    pltpu.sync_copy(dst_hbm.at[pl.ds(base, CH)], dst_v)
    pltpu.sync_copy(sea_hbm.at[pl.ds(base, CH)], sea_v)

    @plsc.parallel_loop(0, CH // LANES, unroll=8)
    def body(i):
        sl = pl.ds(i * LANES, LANES)
        a = plsc.load_gather(sop_v, [src_v[sl]])
        b = plsc.load_gather(smat_v, [dst_v[sl]])
        x = a + b + sea_v[sl]
        out_v[sl] = jnp.maximum(x, 0.2 * x)

    pltpu.sync_copy(out_v, out_hbm.at[pl.ds(base, CH)])

    i16 = lax.iota(jnp.int32, LANES) * EA
    iv = lax.iota(jnp.int32, LANES)
    mj = [(iv + j) % EA for j in range(EA)]
    w0 = wid * (CH * EA)
    pending = []
    for sb in range(NSB):
        pltpu.sync_copy(eabp_hbm.at[pl.ds(w0 + sb * (SB * EA), SB * EA)],
                        pk_v)
        for cp in pending:
            cp.wait()
        pending = []

        @plsc.parallel_loop(0, SB // LANES, unroll=2)
        def fill(i):
            gbase = i * (LANES * EA) + i16
            sbase = i * LANES + iv
            for j in range(EA):
                vals = plsc.load_gather(pk_v, [gbase + mj[j]])
                plsc.store_scatter(eb0, [mj[j] * SB + sbase], vals)

        for k in range(EA):
            cp = pltpu.make_async_copy(
                eb0.at[pl.ds(k * SB, SB)],
                eat_hbm.at[pl.ds(k * E + base + sb * SB, SB)],
                sem0)
            cp.start()
            pending.append(cp)
    for cp in pending:
        cp.wait()


def _cross_logit_stage(s_mat, s_op, src, dst, s_ea, eabp):
    f32, i32 = jnp.float32, jnp.int32
    k = pl.kernel(
        _cross_logit_body,
        out_type=[jax.ShapeDtypeStruct((E,), f32),
                  jax.ShapeDtypeStruct((EA * E,), f32)],
        mesh=plsc.VectorSubcoreMesh(**_SC_MESH),
        compiler_params=_SC_PARAMS,
        scratch_types=[
            pltpu.VMEM((N_MAT,), f32),
            pltpu.VMEM((N_OP,), f32),
            pltpu.VMEM((CH,), i32),
            pltpu.VMEM((CH,), i32),
            pltpu.VMEM((CH,), f32),
            pltpu.VMEM((CH,), f32),
            pltpu.VMEM((SB * EA,), f32),
            pltpu.VMEM((SB * EA,), f32),
            pltpu.SemaphoreType.DMA,
        ],
    )
    return k(s_mat, s_op, src, dst, s_ea, eabp)


def _softmax_body(a_ref, b_ref, wa_ref, wb_ref):
    a = a_ref[...]
    b = b_ref[...]
    m = jnp.maximum(jnp.max(a), jnp.max(b))
    ea_ = jnp.exp(a - m)
    eb_ = jnp.exp(b - m)
    inv = 1.0 / (jnp.sum(ea_) + jnp.sum(eb_))
    wa_ref[...] = ea_ * inv
    wb_ref[...] = eb_ * inv


def _softmax_stage(sself, clog2):
    return pl.pallas_call(
        _softmax_body,
        out_shape=[jax.ShapeDtypeStruct(sself.shape, jnp.float32),
                   jax.ShapeDtypeStruct(clog2.shape, jnp.float32)],
    )(sself, clog2)


def _scatter_body(src_hbm, dst_hbm, wc_hbm, opst_hbm, eat_hbm,
                  acct_hbm, seg16_hbm,
                  t0, t1, t2, t3, a0, a1, a2, a3, acc16,
                  src_v, dst_v, w_v, ea_v):
    wid = lax.axis_index("s") * NC + lax.axis_index("c")
    k_ea = wid % EA
    half = wid // EA
    tbls = (t0, t1, t2, t3)
    accs = (a0, a1, a2, a3)
    for j in range(DPW):
        pltpu.sync_copy(opst_hbm.at[pl.ds((wid * DPW + j) * N_OP, N_OP)],
                        tbls[j])

    zeros = jnp.zeros((LANES,), jnp.float32)
    for j in range(DPW):
        acc_j = accs[j]

        @plsc.parallel_loop(0, N_MAT // LANES, unroll=8)
        def z_body(i):
            acc_j[pl.ds(i * LANES, LANES)] = zeros

    @plsc.parallel_loop(0, N_MAT // LANES, unroll=8)
    def z16_body(i):
        acc16[pl.ds(i * LANES, LANES)] = zeros

    def outer(b, _):
        base = b * EB
        pltpu.sync_copy(src_hbm.at[pl.ds(base, EB)], src_v)
        pltpu.sync_copy(dst_hbm.at[pl.ds(base, EB)], dst_v)
        pltpu.sync_copy(wc_hbm.at[pl.ds(base, EB)], w_v)

        @plsc.parallel_loop(0, EB // LANES, unroll=8)
        def inner(i):
            sl = pl.ds(i * LANES, LANES)
            s16 = src_v[sl]
            d16 = dst_v[sl]
            wv = w_v[sl]
            for j in range(DPW):
                vals = plsc.load_gather(tbls[j], [s16])
                plsc.addupdate_scatter(accs[j], [d16], vals * wv)

        @pl.when((b % 2) == half)
        def _():
            pltpu.sync_copy(eat_hbm.at[pl.ds(k_ea * E + base, EB)], ea_v)

            @plsc.parallel_loop(0, EB // LANES, unroll=8)
            def inner_ea(i):
                sl = pl.ds(i * LANES, LANES)
                d16 = dst_v[sl]
                wv = w_v[sl]
                ev = ea_v[sl]
                plsc.addupdate_scatter(acc16, [d16], ev * wv)

        return 0

    lax.fori_loop(0, NB, outer, 0)
    for j in range(DPW):
        pltpu.sync_copy(accs[j],
                        acct_hbm.at[pl.ds((wid * DPW + j) * N_MAT, N_MAT)])
    pltpu.sync_copy(acc16, seg16_hbm.at[pl.ds(wid * N_MAT, N_MAT)])


def _scatter_stage(src, dst, wc, opst_flat, eat_flat):
    f32, i32 = jnp.float32, jnp.int32
    k = pl.kernel(
        _scatter_body,
        out_type=[jax.ShapeDtypeStruct((EMB * N_MAT,), f32),
                  jax.ShapeDtypeStruct((NW * N_MAT,), f32)],
        mesh=plsc.VectorSubcoreMesh(**_SC_MESH),
        compiler_params=_SC_PARAMS,
        scratch_types=(
            [pltpu.VMEM((N_OP,), f32)] * DPW
            + [pltpu.VMEM((N_MAT,), f32)] * DPW
            + [pltpu.VMEM((N_MAT,), f32),
               pltpu.VMEM((EB,), i32),
               pltpu.VMEM((EB,), i32),
               pltpu.VMEM((EB,), f32),
               pltpu.VMEM((EB,), f32)]
        ),
    )
    return k(src, dst, wc, opst_flat, eat_flat)


def _combine_body(up_ref, wself_ref, acct_ref, seg_ref, w2_ref, out_ref):
    acc = lax.dot_general(acct_ref[...], _eye(EMB), (((0,), (0,)), ((), ())),
                          precision=_HI,
                          preferred_element_type=jnp.float32)
    segt = lax.dot_general(seg_ref[...], _eye(NW), (((0,), (0,)), ((), ())),
                           precision=_HI,
                           preferred_element_type=jnp.float32)
    seg = segt[:, 0:EA] + segt[:, EA:2 * EA]
    term2 = lax.dot_general(seg, w2_ref[...], (((1,), (1,)), ((), ())),
                            precision=_HI,
                            preferred_element_type=jnp.float32)
    x = wself_ref[...] * up_ref[...] + acc + term2
    out_ref[...] = jnp.where(x > 0, x, jnp.exp(jnp.minimum(x, 0.0)) - 1.0)


def _combine_stage(up, wself, acct, seg16p, W2b):
    return pl.pallas_call(
        _combine_body,
        out_shape=jax.ShapeDtypeStruct((N_MAT, EMB), jnp.float32),
    )(up, wself, acct, seg16p, W2b)


def kernel(materials, operations, edge_index, edge_attr,
           W_mat, W_op, att_self_coef, att_coef):
    f32 = jnp.float32
    src = edge_index[0]
    dst = edge_index[1]
    W2 = W_op[:, 112:]
    W2b = W2.astype(jnp.bfloat16).astype(f32)
    eabp = edge_attr.astype(jnp.bfloat16).astype(f32).reshape(-1)
    ops_pad = jnp.pad(operations, ((0, 0), (0, EMB - 112)))

    up, opst, s_self, s_mat, s_op = _node_stage(
        materials, ops_pad, W_mat, W_op, att_self_coef, att_coef)
    s_ea = _sea_stage(edge_attr.reshape(PKR, 128), W2, att_coef)

    clog, eat_flat = _cross_logit_stage(s_mat[:, 0], s_op[:, 0], src, dst,
                                        s_ea.reshape(-1), eabp)

    wself, wc2 = _softmax_stage(s_self, clog.reshape(E // 128, 128))
    wc = wc2.reshape(-1)

    acct_flat, seg16p = _scatter_stage(src, dst, wc,
                                       opst.reshape(-1), eat_flat)
    acct = acct_flat.reshape(EMB, N_MAT)
    seg16p = seg16p.reshape(NW, N_MAT)

    return _combine_stage(up, wself, acct, seg16p, W2b)


# speedup vs baseline: 1.5284x; 1.4553x over previous
import functools

import jax
import jax.numpy as jnp
from jax import lax
from jax.experimental import pallas as pl
from jax.experimental.pallas import tpu as pltpu
from jax.experimental.pallas import tpu_sc as plsc

N_MAT = 10000
N_OP = 10000
E = 320000
EMB = 128
EA = 16

NC = 2
NS = 16
NW = NC * NS

BLK_E = 6400
DPW = EMB // NW
EB = 4000
NB = E // EB
CH = E // NW
LANES = 16

_SC_MESH = dict(core_axis_name="c", subcore_axis_name="s",
                num_cores=NC, num_subcores=NS)
_SC_PARAMS = pltpu.CompilerParams(needs_layout_passes=False)

_HI = lax.Precision.HIGHEST


def _eye(n):
    r = lax.broadcasted_iota(jnp.int32, (n, n), 0)
    c = lax.broadcasted_iota(jnp.int32, (n, n), 1)
    return jnp.where(r == c, 1.0, 0.0).astype(jnp.float32)


def _node_body(mat_ref, ops_ref, wmat_ref, wop_ref, asc_ref, ac_ref,
               up_ref, opst_ref, sself_ref, smat_ref, sop_ref):
    dn = (((1,), (1,)), ((), ()))
    up = lax.dot_general(mat_ref[...], wmat_ref[...], dn,
                         preferred_element_type=jnp.float32)
    opsup = lax.dot_general(ops_ref[...], wop_ref[...], dn,
                            preferred_element_type=jnp.float32)
    up_ref[...] = up
    opst_ref[...] = lax.dot_general(_eye(EMB), opsup, (((1,), (1,)), ((), ())),
                                    precision=_HI,
                                    preferred_element_type=jnp.float32)
    bf = lambda x: x.astype(jnp.bfloat16).astype(jnp.float32)
    upb = bf(up)
    opsupb = bf(opsup)
    c_sum = bf(asc_ref[0:128, 0]) + bf(asc_ref[128:256, 0])
    b1 = bf(ac_ref[0:128, 0])
    b2 = bf(ac_ref[128:256, 0])
    s = jnp.sum(upb * c_sum[None, :], axis=1, keepdims=True)
    sself_ref[...] = jnp.maximum(s, 0.2 * s)
    smat_ref[...] = jnp.sum(upb * b1[None, :], axis=1, keepdims=True)
    sop_ref[...] = jnp.sum(opsupb * b2[None, :], axis=1, keepdims=True)


def _node_stage(materials, ops_pad, W_mat, W_op, asc, ac):
    f32 = jnp.float32
    return pl.pallas_call(
        _node_body,
        out_shape=[
            jax.ShapeDtypeStruct((N_MAT, EMB), f32),
            jax.ShapeDtypeStruct((EMB, N_OP), f32),
            jax.ShapeDtypeStruct((N_MAT, 1), f32),
            jax.ShapeDtypeStruct((N_MAT, 1), f32),
            jax.ShapeDtypeStruct((N_OP, 1), f32),
        ],
    )(materials, ops_pad, W_mat, W_op, asc, ac)


PKR = E * EA // 128
BLK_P = 1600


def _sea_body(pk_ref, w2_ref, ac_ref, sea_ref):
    bf = lambda x: x.astype(jnp.bfloat16).astype(jnp.float32)
    b2 = bf(ac_ref[128:256, 0])
    wv = jnp.sum(b2[:, None] * bf(w2_ref[...]), axis=0)
    wvfull = jnp.concatenate([wv] * 8)
    jj = lax.broadcasted_iota(jnp.int32, (128, 8), 0)
    gg = lax.broadcasted_iota(jnp.int32, (128, 8), 1)
    wv_big = jnp.where(jj // 16 == gg, wvfull[:, None], 0.0)
    sea_ref[...] = lax.dot_general(bf(pk_ref[...]), wv_big,
                                   (((1,), (0,)), ((), ())),
                                   precision=_HI,
                                   preferred_element_type=jnp.float32)


def _sea_stage(ea_packed, W2, ac):
    grid = (PKR // BLK_P,)
    return pl.pallas_call(
        _sea_body,
        grid=grid,
        in_specs=[pl.BlockSpec((BLK_P, 128), lambda i: (i, 0)),
                  pl.BlockSpec((128, EA), lambda i: (0, 0)),
                  pl.BlockSpec((256, 1), lambda i: (0, 0))],
        out_specs=pl.BlockSpec((BLK_P, 8), lambda i: (i, 0)),
        out_shape=jax.ShapeDtypeStruct((PKR, 8), jnp.float32),
    )(ea_packed, W2, ac)


def _cross_logit_body(smat_hbm, sop_hbm, src_hbm, dst_hbm, sea_hbm,
                      out_hbm, smat_v, sop_v, src_v, dst_v, sea_v, out_v):
    wid = lax.axis_index("s") * NC + lax.axis_index("c")
    base = wid * CH
    pltpu.sync_copy(smat_hbm, smat_v)
    pltpu.sync_copy(sop_hbm, sop_v)
    pltpu.sync_copy(src_hbm.at[pl.ds(base, CH)], src_v)
    pltpu.sync_copy(dst_hbm.at[pl.ds(base, CH)], dst_v)
    pltpu.sync_copy(sea_hbm.at[pl.ds(base, CH)], sea_v)

    @plsc.parallel_loop(0, CH // LANES, unroll=8)
    def body(i):
        sl = pl.ds(i * LANES, LANES)
        a = plsc.load_gather(sop_v, [src_v[sl]])
        b = plsc.load_gather(smat_v, [dst_v[sl]])
        x = a + b + sea_v[sl]
        out_v[sl] = jnp.maximum(x, 0.2 * x)

    pltpu.sync_copy(out_v, out_hbm.at[pl.ds(base, CH)])


def _cross_logit_stage(s_mat, s_op, src, dst, s_ea):
    f32, i32 = jnp.float32, jnp.int32
    k = pl.kernel(
        _cross_logit_body,
        out_type=jax.ShapeDtypeStruct((E,), f32),
        mesh=plsc.VectorSubcoreMesh(**_SC_MESH),
        compiler_params=_SC_PARAMS,
        scratch_types=[
            pltpu.VMEM((N_MAT,), f32),
            pltpu.VMEM((N_OP,), f32),
            pltpu.VMEM((CH,), i32),
            pltpu.VMEM((CH,), i32),
            pltpu.VMEM((CH,), f32),
            pltpu.VMEM((CH,), f32),
        ],
    )
    return k(s_mat, s_op, src, dst, s_ea)


def _softmax_body(a_ref, b_ref, wa_ref, wb_ref):
    a = a_ref[...]
    b = b_ref[...]
    m = jnp.maximum(jnp.max(a), jnp.max(b))
    ea_ = jnp.exp(a - m)
    eb_ = jnp.exp(b - m)
    inv = 1.0 / (jnp.sum(ea_) + jnp.sum(eb_))
    wa_ref[...] = ea_ * inv
    wb_ref[...] = eb_ * inv


def _softmax_stage(sself, clog2):
    return pl.pallas_call(
        _softmax_body,
        out_shape=[jax.ShapeDtypeStruct(sself.shape, jnp.float32),
                   jax.ShapeDtypeStruct(clog2.shape, jnp.float32)],
    )(sself, clog2)


def _scatter_body(src_hbm, dst_hbm, wc_hbm, opst_hbm, eat_hbm,
                  acct_hbm, seg16_hbm,
                  t0, t1, t2, t3, a0, a1, a2, a3, acc16,
                  sA, dA, wA, eA, sB, dB, wB, eB, semA, semB):
    wid = lax.axis_index("s") * NC + lax.axis_index("c")
    k_ea = wid % EA
    half = wid // EA
    tbls = (t0, t1, t2, t3)
    accs = (a0, a1, a2, a3)
    bufA = (sA, dA, wA, eA)
    bufB = (sB, dB, wB, eB)
    for j in range(DPW):
        pltpu.sync_copy(opst_hbm.at[pl.ds((wid * DPW + j) * N_OP, N_OP)],
                        tbls[j])

    zeros = jnp.zeros((LANES,), jnp.float32)
    for j in range(DPW):
        acc_j = accs[j]

        @plsc.parallel_loop(0, N_MAT // LANES, unroll=8)
        def z_body(i):
            acc_j[pl.ds(i * LANES, LANES)] = zeros

    @plsc.parallel_loop(0, N_MAT // LANES, unroll=8)
    def z16_body(i):
        acc16[pl.ds(i * LANES, LANES)] = zeros

    def stage(b, bufs, sem):
        base = b * EB
        cps = [
            pltpu.make_async_copy(src_hbm.at[pl.ds(base, EB)], bufs[0], sem),
            pltpu.make_async_copy(dst_hbm.at[pl.ds(base, EB)], bufs[1], sem),
            pltpu.make_async_copy(wc_hbm.at[pl.ds(base, EB)], bufs[2], sem),
            pltpu.make_async_copy(eat_hbm.at[pl.ds(k_ea * E + base, EB)],
                                  bufs[3], sem),
        ]
        for cp in cps:
            cp.start()
        return cps

    def compute(b, bufs):
        src_v, dst_v, w_v, ea_v = bufs

        @plsc.parallel_loop(0, EB // LANES, unroll=8)
        def inner(i):
            sl = pl.ds(i * LANES, LANES)
            s16 = src_v[sl]
            d16 = dst_v[sl]
            wv = w_v[sl]
            for j in range(DPW):
                vals = plsc.load_gather(tbls[j], [s16])
                plsc.addupdate_scatter(accs[j], [d16], vals * wv)

        @pl.when((b % 2) == half)
        def _():
            @plsc.parallel_loop(0, EB // LANES, unroll=8)
            def inner_ea(i):
                sl = pl.ds(i * LANES, LANES)
                d16 = dst_v[sl]
                wv = w_v[sl]
                ev = ea_v[sl]
                plsc.addupdate_scatter(acc16, [d16], ev * wv)

    for cp in stage(0, bufA, semA):
        cp.wait()

    def outer(t, _):
        b0 = t * 2
        cpsB = stage(b0 + 1, bufB, semB)
        compute(b0, bufA)
        for cp in cpsB:
            cp.wait()
        cpsA = stage(lax.rem(b0 + 2, NB), bufA, semA)
        compute(b0 + 1, bufB)
        for cp in cpsA:
            cp.wait()
        return 0

    lax.fori_loop(0, NB // 2, outer, 0)
    for j in range(DPW):
        pltpu.sync_copy(accs[j],
                        acct_hbm.at[pl.ds((wid * DPW + j) * N_MAT, N_MAT)])
    pltpu.sync_copy(acc16, seg16_hbm.at[pl.ds(wid * N_MAT, N_MAT)])


def _scatter_stage(src, dst, wc, opst_flat, eat_flat):
    f32, i32 = jnp.float32, jnp.int32
    ebufs = [pltpu.VMEM((EB,), i32), pltpu.VMEM((EB,), i32),
             pltpu.VMEM((EB,), f32), pltpu.VMEM((EB,), f32)]
    k = pl.kernel(
        _scatter_body,
        out_type=[jax.ShapeDtypeStruct((EMB * N_MAT,), f32),
                  jax.ShapeDtypeStruct((NW * N_MAT,), f32)],
        mesh=plsc.VectorSubcoreMesh(**_SC_MESH),
        compiler_params=_SC_PARAMS,
        scratch_types=(
            [pltpu.VMEM((N_OP,), f32)] * DPW
            + [pltpu.VMEM((N_MAT,), f32)] * DPW
            + [pltpu.VMEM((N_MAT,), f32)]
            + ebufs + ebufs
            + [pltpu.SemaphoreType.DMA, pltpu.SemaphoreType.DMA]
        ),
    )
    return k(src, dst, wc, opst_flat, eat_flat)


def _combine_body(up_ref, wself_ref, acct_ref, seg_ref, w2_ref, out_ref):
    acc = lax.dot_general(acct_ref[...], _eye(EMB), (((0,), (0,)), ((), ())),
                          precision=_HI,
                          preferred_element_type=jnp.float32)
    segt = lax.dot_general(seg_ref[...], _eye(NW), (((0,), (0,)), ((), ())),
                           precision=_HI,
                           preferred_element_type=jnp.float32)
    seg = segt[:, 0:EA] + segt[:, EA:2 * EA]
    term2 = lax.dot_general(seg, w2_ref[...], (((1,), (1,)), ((), ())),
                            precision=_HI,
                            preferred_element_type=jnp.float32)
    x = wself_ref[...] * up_ref[...] + acc + term2
    out_ref[...] = jnp.where(x > 0, x, jnp.exp(jnp.minimum(x, 0.0)) - 1.0)


def _combine_stage(up, wself, acct, seg16p, W2b):
    return pl.pallas_call(
        _combine_body,
        out_shape=jax.ShapeDtypeStruct((N_MAT, EMB), jnp.float32),
    )(up, wself, acct, seg16p, W2b)


def kernel(materials, operations, edge_index, edge_attr,
           W_mat, W_op, att_self_coef, att_coef):
    f32 = jnp.float32
    src = edge_index[0]
    dst = edge_index[1]
    W2 = W_op[:, 112:]
    W2b = W2.astype(jnp.bfloat16).astype(f32)
    eat_flat = edge_attr.astype(jnp.bfloat16).astype(f32).T.reshape(-1)
    ops_pad = jnp.pad(operations, ((0, 0), (0, EMB - 112)))

    up, opst, s_self, s_mat, s_op = _node_stage(
        materials, ops_pad, W_mat, W_op, att_self_coef, att_coef)
    s_ea = _sea_stage(edge_attr.reshape(PKR, 128), W2, att_coef)

    clog = _cross_logit_stage(s_mat[:, 0], s_op[:, 0], src, dst,
                              s_ea.reshape(-1))

    wself, wc2 = _softmax_stage(s_self, clog.reshape(E // 128, 128))
    wc = wc2.reshape(-1)

    acct_flat, seg16p = _scatter_stage(src, dst, wc,
                                       opst.reshape(-1), eat_flat)
    acct = acct_flat.reshape(EMB, N_MAT)
    seg16p = seg16p.reshape(NW, N_MAT)

    return _combine_stage(up, wself, acct, seg16p, W2b)


# speedup vs baseline: 1.5286x; 1.0001x over previous
import jax
import jax.numpy as jnp
from jax import lax
from jax.experimental import pallas as pl
from jax.experimental.pallas import tpu as pltpu
from jax.experimental.pallas import tpu_sc as plsc

N_MAT = 10000
N_OP = 10000
E = 320000
EMB = 128
EA = 16

NC = 2
NS = 16
NW = NC * NS

DPW = EMB // NW
EB = 4000
NB = E // EB
CH = E // NW
LANES = 16

_SC_MESH = dict(core_axis_name="c", subcore_axis_name="s",
                num_cores=NC, num_subcores=NS)
_SC_PARAMS = pltpu.CompilerParams(needs_layout_passes=False)

_HI = lax.Precision.HIGHEST


def _eye(n):
    r = lax.broadcasted_iota(jnp.int32, (n, n), 0)
    c = lax.broadcasted_iota(jnp.int32, (n, n), 1)
    return jnp.where(r == c, 1.0, 0.0).astype(jnp.float32)


def _node_body(mat_ref, ops_ref, wmat_ref, wop_ref, asc_ref, ac_ref,
               up_ref, opst_ref, sself_ref, smat_ref, sop_ref):
    dn = (((1,), (1,)), ((), ()))
    up = lax.dot_general(mat_ref[...], wmat_ref[...], dn,
                         preferred_element_type=jnp.float32)
    opsup = lax.dot_general(ops_ref[...], wop_ref[...], dn,
                            preferred_element_type=jnp.float32)
    up_ref[...] = up
    opst_ref[...] = lax.dot_general(_eye(EMB), opsup, (((1,), (1,)), ((), ())),
                                    precision=_HI,
                                    preferred_element_type=jnp.float32)
    bf = lambda x: x.astype(jnp.bfloat16).astype(jnp.float32)
    upb = bf(up)
    opsupb = bf(opsup)
    c_sum = bf(asc_ref[0:128, 0]) + bf(asc_ref[128:256, 0])
    b1 = bf(ac_ref[0:128, 0])
    b2 = bf(ac_ref[128:256, 0])
    s = jnp.sum(upb * c_sum[None, :], axis=1, keepdims=True)
    sself_ref[...] = jnp.maximum(s, 0.2 * s)
    smat_ref[...] = jnp.sum(upb * b1[None, :], axis=1, keepdims=True)
    sop_ref[...] = jnp.sum(opsupb * b2[None, :], axis=1, keepdims=True)


def _node_stage(materials, ops_pad, W_mat, W_op, asc, ac):
    f32 = jnp.float32
    return pl.pallas_call(
        _node_body,
        out_shape=[
            jax.ShapeDtypeStruct((N_MAT, EMB), f32),
            jax.ShapeDtypeStruct((EMB, N_OP), f32),
            jax.ShapeDtypeStruct((N_MAT, 1), f32),
            jax.ShapeDtypeStruct((N_MAT, 1), f32),
            jax.ShapeDtypeStruct((N_OP, 1), f32),
        ],
    )(materials, ops_pad, W_mat, W_op, asc, ac)


PKR = E * EA // 128
BLK_P = 1600


def _sea_body(pk_ref, w2_ref, ac_ref, sea_ref):
    bf = lambda x: x.astype(jnp.bfloat16).astype(jnp.float32)
    b2 = bf(ac_ref[128:256, 0])
    wv = jnp.sum(b2[:, None] * bf(w2_ref[...]), axis=0)
    wvfull = jnp.concatenate([wv] * 8)
    jj = lax.broadcasted_iota(jnp.int32, (128, 8), 0)
    gg = lax.broadcasted_iota(jnp.int32, (128, 8), 1)
    wv_big = jnp.where(jj // 16 == gg, wvfull[:, None], 0.0)
    sea_ref[...] = lax.dot_general(bf(pk_ref[...]), wv_big,
                                   (((1,), (0,)), ((), ())),
                                   precision=_HI,
                                   preferred_element_type=jnp.float32)


def _sea_stage(ea_packed, W2, ac):
    grid = (PKR // BLK_P,)
    return pl.pallas_call(
        _sea_body,
        grid=grid,
        in_specs=[pl.BlockSpec((BLK_P, 128), lambda i: (i, 0)),
                  pl.BlockSpec((128, EA), lambda i: (0, 0)),
                  pl.BlockSpec((256, 1), lambda i: (0, 0))],
        out_specs=pl.BlockSpec((BLK_P, 8), lambda i: (i, 0)),
        out_shape=jax.ShapeDtypeStruct((PKR, 8), jnp.float32),
    )(ea_packed, W2, ac)


def _cross_logit_body(smat_hbm, sop_hbm, src_hbm, dst_hbm, sea_hbm,
                      out_hbm, smat_v, sop_v, src_v, dst_v, sea_v, out_v):
    wid = lax.axis_index("s") * NC + lax.axis_index("c")
    base = wid * CH
    pltpu.sync_copy(smat_hbm, smat_v)
    pltpu.sync_copy(sop_hbm, sop_v)
    pltpu.sync_copy(src_hbm.at[pl.ds(base, CH)], src_v)
    pltpu.sync_copy(dst_hbm.at[pl.ds(base, CH)], dst_v)
    pltpu.sync_copy(sea_hbm.at[pl.ds(base, CH)], sea_v)

    @plsc.parallel_loop(0, CH // LANES, unroll=8)
    def body(i):
        sl = pl.ds(i * LANES, LANES)
        a = plsc.load_gather(sop_v, [src_v[sl]])
        b = plsc.load_gather(smat_v, [dst_v[sl]])
        x = a + b + sea_v[sl]
        out_v[sl] = jnp.maximum(x, 0.2 * x)

    pltpu.sync_copy(out_v, out_hbm.at[pl.ds(base, CH)])


def _cross_logit_stage(s_mat, s_op, src, dst, s_ea):
    f32, i32 = jnp.float32, jnp.int32
    k = pl.kernel(
        _cross_logit_body,
        out_type=jax.ShapeDtypeStruct((E,), f32),
        mesh=plsc.VectorSubcoreMesh(**_SC_MESH),
        compiler_params=_SC_PARAMS,
        scratch_types=[
            pltpu.VMEM((N_MAT,), f32),
            pltpu.VMEM((N_OP,), f32),
            pltpu.VMEM((CH,), i32),
            pltpu.VMEM((CH,), i32),
            pltpu.VMEM((CH,), f32),
            pltpu.VMEM((CH,), f32),
        ],
    )
    return k(s_mat, s_op, src, dst, s_ea)


def _softmax_body(a_ref, b_ref, wa_ref, wb_ref):
    a = a_ref[...]
    b = b_ref[...]
    m = jnp.maximum(jnp.max(a), jnp.max(b))
    ea_ = jnp.exp(a - m)
    eb_ = jnp.exp(b - m)
    inv = 1.0 / (jnp.sum(ea_) + jnp.sum(eb_))
    wa_ref[...] = ea_ * inv
    wb_ref[...] = eb_ * inv


def _softmax_stage(sself, clog2):
    return pl.pallas_call(
        _softmax_body,
        out_shape=[jax.ShapeDtypeStruct(sself.shape, jnp.float32),
                   jax.ShapeDtypeStruct(clog2.shape, jnp.float32)],
    )(sself, clog2)


def _scatter_body(src_hbm, dst_hbm, wc_hbm, opst_hbm, eat_hbm,
                  acct_hbm, seg16_hbm,
                  t0, t1, t2, t3, a0, a1, a2, a3, acc16,
                  sA, dA, wA, eA, sB, dB, wB, eB, semA, semB):
    wid = lax.axis_index("s") * NC + lax.axis_index("c")
    k_ea = wid % EA
    half = wid // EA
    tbls = (t0, t1, t2, t3)
    accs = (a0, a1, a2, a3)
    bufA = (sA, dA, wA, eA)
    bufB = (sB, dB, wB, eB)
    for j in range(DPW):
        pltpu.sync_copy(opst_hbm.at[pl.ds((wid * DPW + j) * N_OP, N_OP)],
                        tbls[j])

    zeros = jnp.zeros((LANES,), jnp.float32)
    for j in range(DPW):
        acc_j = accs[j]

        @plsc.parallel_loop(0, N_MAT // LANES, unroll=8)
        def z_body(i):
            acc_j[pl.ds(i * LANES, LANES)] = zeros

    @plsc.parallel_loop(0, N_MAT // LANES, unroll=8)
    def z16_body(i):
        acc16[pl.ds(i * LANES, LANES)] = zeros

    def stage(b, bufs, sem):
        base = b * EB
        cps = [
            pltpu.make_async_copy(src_hbm.at[pl.ds(base, EB)], bufs[0], sem),
            pltpu.make_async_copy(dst_hbm.at[pl.ds(base, EB)], bufs[1], sem),
            pltpu.make_async_copy(wc_hbm.at[pl.ds(base, EB)], bufs[2], sem),
            pltpu.make_async_copy(eat_hbm.at[pl.ds(k_ea * E + base, EB)],
                                  bufs[3], sem),
        ]
        for cp in cps:
            cp.start()
        return cps

    def compute(b, bufs):
        src_v, dst_v, w_v, ea_v = bufs

        @plsc.parallel_loop(0, EB // LANES, unroll=8)
        def inner(i):
            sl = pl.ds(i * LANES, LANES)
            s16 = src_v[sl]
            d16 = dst_v[sl]
            wv = w_v[sl]
            for j in range(DPW):
                vals = plsc.load_gather(tbls[j], [s16])
                plsc.addupdate_scatter(accs[j], [d16], vals * wv)

        @pl.when((b % 2) == half)
        def _():
            @plsc.parallel_loop(0, EB // LANES, unroll=8)
            def inner_ea(i):
                sl = pl.ds(i * LANES, LANES)
                d16 = dst_v[sl]
                wv = w_v[sl]
                ev = ea_v[sl]
                plsc.addupdate_scatter(acc16, [d16], ev * wv)

    for cp in stage(0, bufA, semA):
        cp.wait()

    def outer(t, _):
        b0 = t * 2
        cpsB = stage(b0 + 1, bufB, semB)
        compute(b0, bufA)
        for cp in cpsB:
            cp.wait()
        cpsA = stage(lax.rem(b0 + 2, NB), bufA, semA)
        compute(b0 + 1, bufB)
        for cp in cpsA:
            cp.wait()
        return 0

    lax.fori_loop(0, NB // 2, outer, 0)
    for j in range(DPW):
        pltpu.sync_copy(accs[j],
                        acct_hbm.at[pl.ds((wid * DPW + j) * N_MAT, N_MAT)])
    pltpu.sync_copy(acc16, seg16_hbm.at[pl.ds(wid * N_MAT, N_MAT)])


def _scatter_stage(src, dst, wc, opst_flat, eat_flat):
    f32, i32 = jnp.float32, jnp.int32
    ebufs = [pltpu.VMEM((EB,), i32), pltpu.VMEM((EB,), i32),
             pltpu.VMEM((EB,), f32), pltpu.VMEM((EB,), f32)]
    k = pl.kernel(
        _scatter_body,
        out_type=[jax.ShapeDtypeStruct((EMB * N_MAT,), f32),
                  jax.ShapeDtypeStruct((NW * N_MAT,), f32)],
        mesh=plsc.VectorSubcoreMesh(**_SC_MESH),
        compiler_params=_SC_PARAMS,
        scratch_types=(
            [pltpu.VMEM((N_OP,), f32)] * DPW
            + [pltpu.VMEM((N_MAT,), f32)] * DPW
            + [pltpu.VMEM((N_MAT,), f32)]
            + ebufs + ebufs
            + [pltpu.SemaphoreType.DMA, pltpu.SemaphoreType.DMA]
        ),
    )
    return k(src, dst, wc, opst_flat, eat_flat)


def _combine_body(up_ref, wself_ref, acct_ref, seg_ref, w2_ref, out_ref):
    acc = lax.dot_general(acct_ref[...], _eye(EMB), (((0,), (0,)), ((), ())),
                          precision=_HI,
                          preferred_element_type=jnp.float32)
    segt = lax.dot_general(seg_ref[...], _eye(NW), (((0,), (0,)), ((), ())),
                           precision=_HI,
                           preferred_element_type=jnp.float32)
    seg = segt[:, 0:EA] + segt[:, EA:2 * EA]
    term2 = lax.dot_general(seg, w2_ref[...], (((1,), (1,)), ((), ())),
                            precision=_HI,
                            preferred_element_type=jnp.float32)
    x = wself_ref[...] * up_ref[...] + acc + term2
    out_ref[...] = jnp.where(x > 0, x, jnp.exp(jnp.minimum(x, 0.0)) - 1.0)


def _combine_stage(up, wself, acct, seg16p, W2b):
    return pl.pallas_call(
        _combine_body,
        out_shape=jax.ShapeDtypeStruct((N_MAT, EMB), jnp.float32),
    )(up, wself, acct, seg16p, W2b)


def kernel(materials, operations, edge_index, edge_attr,
           W_mat, W_op, att_self_coef, att_coef):
    f32 = jnp.float32
    src = edge_index[0]
    dst = edge_index[1]
    W2 = W_op[:, 112:]
    W2b = W2.astype(jnp.bfloat16).astype(f32)
    eat_flat = edge_attr.astype(jnp.bfloat16).astype(f32).T.reshape(-1)
    ops_pad = jnp.pad(operations, ((0, 0), (0, EMB - 112)))

    up, opst, s_self, s_mat, s_op = _node_stage(
        materials, ops_pad, W_mat, W_op, att_self_coef, att_coef)
    s_ea = _sea_stage(edge_attr.reshape(PKR, 128), W2, att_coef)

    clog = _cross_logit_stage(s_mat[:, 0], s_op[:, 0], src, dst,
                              s_ea.reshape(-1))

    wself, wc2 = _softmax_stage(s_self, clog.reshape(E // 128, 128))
    wc = wc2.reshape(-1)

    acct_flat, seg16p = _scatter_stage(src, dst, wc,
                                       opst.reshape(-1), eat_flat)
    acct = acct_flat.reshape(EMB, N_MAT)
    seg16p = seg16p.reshape(NW, N_MAT)

    return _combine_stage(up, wself, acct, seg16p, W2b)
